# baseline (speedup 1.0000x reference)
import functools
import jax
import jax.numpy as jnp
from jax import lax
from jax.experimental import pallas as pl
from jax.experimental.pallas import tpu as pltpu
from jax.experimental.pallas import tpu_sc as plsc

N = 100000
E = 1600000
IN_DIM = 4
HID = 64

NC = 2
NS = 16
NW = NC * NS

N_PAD = 100352
NSL = N_PAD // NS
SAC = N

EPT = 50048
E_PAD = NW * EPT
CHE = 2944
NCHUNK = EPT // CHE
CHB = CHE // 128

f32 = jnp.float32


def _worker_ids():
    cid = lax.axis_index("c")
    sid = lax.axis_index("s")
    return cid, sid, cid * NS + sid


def _sc_deg_body(srcR, dstR, zeros_n, ones2,
                 degf_out, degu_out,
                 degf_sh, degu_sh,
                 s_v0, s_v1, s_v2, d_v0, d_v1, d_v2, ones_v,
                 sem_i, sem_s0, sem_s1, sem_s2, sem_st):
    s_v = (s_v0, s_v1, s_v2)
    d_v = (d_v0, d_v1, d_v2)
    sem_s = (sem_s0, sem_s1, sem_s2)
    cid, sid, wid = _worker_ids()
    off = sid * NSL
    sl = pl.ds(off, NSL)
    stage = [pltpu.async_copy(zeros_n.at[sl], degf_sh.at[sl], sem_st),
             pltpu.async_copy(zeros_n.at[sl], degu_sh.at[sl], sem_st),
             pltpu.async_copy(ones2, ones_v, sem_st)]
    for d in stage:
        d.wait()
    plsc.subcore_barrier()
    e0 = wid * EPT

    idx_pend = [None] * 3
    sca_pend = [None] * 3

    def start_idx(c):
        r = e0 + c * CHE
        m = c % 3
        idx_pend[m] = [
            pltpu.async_copy(srcR.at[pl.ds(r, CHE)], s_v[m], sem_i),
            pltpu.async_copy(dstR.at[pl.ds(r, CHE)], d_v[m], sem_i)]

    start_idx(0)
    for c in range(NCHUNK):
        m = c % 3
        for d in idx_pend[m]:
            d.wait()
        if c >= 2 and sca_pend[(c - 2) % 3]:
            for d in sca_pend[(c - 2) % 3]:
                d.wait()
            sca_pend[(c - 2) % 3] = None
        if c + 1 < NCHUNK:
            start_idx(c + 1)
        sca_pend[m] = []
        for dstref, idxref in ((degf_sh, d_v[m]), (degu_sh, s_v[m])):
            dd = pltpu.make_async_copy(ones_v, dstref.at[idxref], sem_s[m])
            dd.start(add=True)
            sca_pend[m].append(dd)
    for m in range(3):
        if sca_pend[m]:
            for d in sca_pend[m]:
                d.wait()
    plsc.subcore_barrier()
    out = [pltpu.async_copy(degf_sh.at[sl], degf_out.at[cid, sl], sem_st),
           pltpu.async_copy(degu_sh.at[sl], degu_out.at[cid, sl], sem_st)]
    for d in out:
        d.wait()


_sc_deg = pl.kernel(
    _sc_deg_body,
    out_type=[jax.ShapeDtypeStruct((NC, N_PAD), f32),
              jax.ShapeDtypeStruct((NC, N_PAD), f32)],
    mesh=plsc.VectorSubcoreMesh(core_axis_name="c", subcore_axis_name="s"),
    scratch_types=(
        [pltpu.VMEM_SHARED((N_PAD,), f32)] * 2
        + [pltpu.VMEM((CHE,), jnp.int32)] * 6
        + [pltpu.VMEM((CHE,), f32)]
        + [pltpu.SemaphoreType.DMA] * 5
    ),
)


def _sc_agg_body(aR, bR2, t_hbm, zeros_nm,
                 acc_out,
                 t_sh, acc_sh,
                 a_v0, a_v1, a_v2, b_v0, b_v1, b_v2,
                 val0, val1,
                 sem_i, sem_g, sem_s0, sem_s1, sem_st):
    a_v = (a_v0, a_v1, a_v2)
    b_v = (b_v0, b_v1, b_v2)
    val = (val0, val1)
    sem_s = (sem_s0, sem_s1)
    cid, sid, wid = _worker_ids()
    off = sid * NSL
    sl = pl.ds(off, NSL)
    stage = [pltpu.async_copy(t_hbm.at[sl], t_sh.at[sl], sem_st),
             pltpu.async_copy(zeros_nm.at[sl], acc_sh.at[sl], sem_st)]
    for d in stage:
        d.wait()
    plsc.subcore_barrier()
    e0 = wid * EPT

    idx_pend = [None] * 3
    sca_pend = [None, None]

    r0 = wid * (EPT // 128)

    def start_idx(c):
        r = e0 + c * CHE
        m = c % 3
        idx_pend[m] = [
            pltpu.async_copy(aR.at[pl.ds(r, CHE)], a_v[m], sem_i),
            pltpu.async_copy(bR2.at[pl.ds(r0 + c * CHB, CHB), :], b_v[m], sem_i)]

    start_idx(0)
    for c in range(NCHUNK):
        m = c % 3
        p = c & 1
        for d in idx_pend[m]:
            d.wait()
        if sca_pend[p]:
            for d in sca_pend[p]:
                d.wait()
            sca_pend[p] = None
        if c + 1 < NCHUNK:
            start_idx(c + 1)
        pltpu.async_copy(t_sh.at[a_v[m]], val[p], sem_g).wait()
        sca_pend[p] = []
        for j in range(CHB):
            dd = pltpu.make_async_copy(val[p].at[pl.ds(j * 128, 128)],
                                       acc_sh.at[b_v[m].at[j]], sem_s[p])
            dd.start(add=True)
            sca_pend[p].append(dd)
    for p in (0, 1):
        if sca_pend[p]:
            for d in sca_pend[p]:
                d.wait()
    plsc.subcore_barrier()
    pltpu.async_copy(acc_sh.at[sl], acc_out.at[cid, sl], sem_st).wait()


_sc_agg = pl.kernel(
    _sc_agg_body,
    out_type=jax.ShapeDtypeStruct((NC, N_PAD, IN_DIM), f32),
    mesh=plsc.VectorSubcoreMesh(core_axis_name="c", subcore_axis_name="s"),
    scratch_types=(
        [pltpu.VMEM_SHARED((N_PAD, IN_DIM), f32)] * 2
        + [pltpu.VMEM((CHE,), jnp.int32)] * 3
        + [pltpu.VMEM((CHB, 128), jnp.int32)] * 3
        + [pltpu.VMEM((CHE, IN_DIM), f32)] * 2
        + [pltpu.SemaphoreType.DMA] * 5
    ),
    compiler_params=pltpu.CompilerParams(use_tc_tiling_on_sc=False),
)


def _sc_y_body(srcR, dstR, ys_hbm, zeros_n,
               ay_out,
               ys_sh, ay_sh,
               s_v0, s_v1, s_v2, d_v0, d_v1, d_v2, r1_v0, r1_v1,
               sem_i, sem_g, sem_s0, sem_s1, sem_st):
    s_v = (s_v0, s_v1, s_v2)
    d_v = (d_v0, d_v1, d_v2)
    r1 = (r1_v0, r1_v1)
    sem_s = (sem_s0, sem_s1)
    cid, sid, wid = _worker_ids()
    off = sid * NSL
    sl = pl.ds(off, NSL)
    stage = [pltpu.async_copy(ys_hbm.at[sl], ys_sh.at[sl], sem_st),
             pltpu.async_copy(zeros_n.at[sl], ay_sh.at[sl], sem_st)]
    for d in stage:
        d.wait()
    plsc.subcore_barrier()
    e0 = wid * EPT

    idx_pend = [None] * 3
    sca_pend = [None, None]

    def start_idx(c):
        r = e0 + c * CHE
        m = c % 3
        idx_pend[m] = [
            pltpu.async_copy(srcR.at[pl.ds(r, CHE)], s_v[m], sem_i),
            pltpu.async_copy(dstR.at[pl.ds(r, CHE)], d_v[m], sem_i)]

    start_idx(0)
    for c in range(NCHUNK):
        m = c % 3
        p = c & 1
        for d in idx_pend[m]:
            d.wait()
        if sca_pend[p]:
            for d in sca_pend[p]:
                d.wait()
            sca_pend[p] = None
        if c + 1 < NCHUNK:
            start_idx(c + 1)
        pltpu.async_copy(ys_sh.at[s_v[m]], r1[p], sem_g).wait()
        dd = pltpu.make_async_copy(r1[p], ay_sh.at[d_v[m]], sem_s[p])
        dd.start(add=True)
        sca_pend[p] = [dd]
    for p in (0, 1):
        if sca_pend[p]:
            for d in sca_pend[p]:
                d.wait()
    plsc.subcore_barrier()
    pltpu.async_copy(ay_sh.at[sl], ay_out.at[cid, sl], sem_st).wait()


_sc_y = pl.kernel(
    _sc_y_body,
    out_type=jax.ShapeDtypeStruct((NC, N_PAD), f32),
    mesh=plsc.VectorSubcoreMesh(core_axis_name="c", subcore_axis_name="s"),
    scratch_types=(
        [pltpu.VMEM_SHARED((N_PAD,), f32)] * 2
        + [pltpu.VMEM((CHE,), jnp.int32)] * 6
        + [pltpu.VMEM((CHE,), f32)] * 2
        + [pltpu.SemaphoreType.DMA] * 5
    ),
)


def _tc_norm_body(degf_ref, degu_ref, xT_ref,
                  df_ref, du_ref, pT_ref, qT_ref):
    degf = degf_ref[0:1, :] + degf_ref[1:2, :] + 1.0
    degu = degu_ref[0:1, :] + degu_ref[1:2, :] + 1.0
    df = 1.0 / jnp.sqrt(degf)
    du = 1.0 / jnp.sqrt(degu)
    df_ref[...] = df
    du_ref[...] = du
    pT_ref[...] = xT_ref[...] * df
    qT_ref[...] = xT_ref[...] * du


_tc_norm = pl.pallas_call(
    _tc_norm_body,
    out_shape=[jax.ShapeDtypeStruct((1, N_PAD), f32),
               jax.ShapeDtypeStruct((1, N_PAD), f32),
               jax.ShapeDtypeStruct((IN_DIM, N_PAD), f32),
               jax.ShapeDtypeStruct((IN_DIM, N_PAD), f32)],
)

BN = 2048
GRID_B = N_PAD // BN


def _mm4(wT, a):
    acc = wT[:, 0:1] * a[0:1, :]
    for k in range(1, IN_DIM):
        acc += wT[:, k:k + 1] * a[k:k + 1, :]
    return acc


def _tc_dense_body(afT_ref, auT_ref, pT_ref, qT_ref, df_ref, du_ref,
                   wfT_ref, wuT_ref, bf_ref, bu_ref,
                   wfc1T_ref, wfc2T_ref, bfc_ref, wo_ref,
                   ys_ref):
    af = (afT_ref[0] + afT_ref[1] + pT_ref[...]) * df_ref[...]
    au = (auT_ref[0] + auT_ref[1] + qT_ref[...]) * du_ref[...]
    hf = jnp.maximum(_mm4(wfT_ref[...], af) + bf_ref[...], 0.0)
    hu = jnp.maximum(_mm4(wuT_ref[...], au) + bu_ref[...], 0.0)
    h2 = jnp.dot(wfc1T_ref[...], hf, preferred_element_type=f32)
    h2 += jnp.dot(wfc2T_ref[...], hu, preferred_element_type=f32)
    h2 = jnp.maximum(h2 + bfc_ref[...], 0.0)
    y = jnp.sum(h2 * wo_ref[...], axis=0, keepdims=True)
    ys_ref[...] = y * df_ref[...]


_tc_dense = pl.pallas_call(
    _tc_dense_body,
    grid=(N_PAD // BN,),
    in_specs=[
        pl.BlockSpec((NC, IN_DIM, BN), lambda i: (0, 0, i)),
        pl.BlockSpec((NC, IN_DIM, BN), lambda i: (0, 0, i)),
        pl.BlockSpec((IN_DIM, BN), lambda i: (0, i)),
        pl.BlockSpec((IN_DIM, BN), lambda i: (0, i)),
        pl.BlockSpec((1, BN), lambda i: (0, i)),
        pl.BlockSpec((1, BN), lambda i: (0, i)),
        pl.BlockSpec((HID, IN_DIM), lambda i: (0, 0)),
        pl.BlockSpec((HID, IN_DIM), lambda i: (0, 0)),
        pl.BlockSpec((HID, 1), lambda i: (0, 0)),
        pl.BlockSpec((HID, 1), lambda i: (0, 0)),
        pl.BlockSpec((HID, HID), lambda i: (0, 0)),
        pl.BlockSpec((HID, HID), lambda i: (0, 0)),
        pl.BlockSpec((HID, 1), lambda i: (0, 0)),
        pl.BlockSpec((HID, 1), lambda i: (0, 0)),
    ],
    out_specs=pl.BlockSpec((1, BN), lambda i: (0, i)),
    out_shape=jax.ShapeDtypeStruct((1, N_PAD), f32),
)


def _tc_out_body(ay_ref, ys_ref, df_ref, bo_ref, out_ref):
    s = df_ref[...] * (ay_ref[0:1, :] + ay_ref[1:2, :] + ys_ref[...]) + bo_ref[0, 0]
    out_ref[...] = jax.nn.sigmoid(s)


_tc_out = pl.pallas_call(
    _tc_out_body,
    out_shape=jax.ShapeDtypeStruct((1, N_PAD), f32),
)


@jax.jit
def kernel(x, edge_index, W_f, b_f, W_u, b_u, W_fc, b_fc, W_o, b_o):
    src = edge_index[0].astype(jnp.int32)
    dst = edge_index[1].astype(jnp.int32)
    padi = jnp.full((E_PAD - E,), SAC, dtype=jnp.int32)
    srcR = jnp.concatenate([src, padi])
    dstR = jnp.concatenate([dst, padi])

    zeros_n = jnp.zeros((N_PAD,), f32)
    ones2 = jnp.ones((CHE,), f32)

    degf_p, degu_p = _sc_deg(srcR, dstR, zeros_n, ones2)

    xT = jnp.zeros((IN_DIM, N_PAD), f32).at[:, :N].set(x.T)
    df, du, pT, qT = _tc_norm(degf_p, degu_p, xT)

    zeros_nm = jnp.zeros((N_PAD, IN_DIM), f32)
    srcR2 = srcR.reshape(-1, 128)
    dstR2 = dstR.reshape(-1, 128)
    af_p = _sc_agg(srcR, dstR2, pT.T, zeros_nm)
    au_p = _sc_agg(dstR, srcR2, qT.T, zeros_nm)

    ysT = _tc_dense(af_p.transpose(0, 2, 1), au_p.transpose(0, 2, 1),
                    pT, qT, df, du,
                    W_f.T, W_u.T, b_f[:, None], b_u[:, None],
                    W_fc[:HID].T, W_fc[HID:].T, b_fc[:, None], W_o)

    ay_p = _sc_y(srcR, dstR, ysT[0], zeros_n)

    outT = _tc_out(ay_p, ysT, df, b_o.reshape(1, 1))
    return outT[0, :N][:, None]


# speedup vs baseline: 1.3200x; 1.3200x over previous
import functools
import jax
import jax.numpy as jnp
from jax import lax
from jax.experimental import pallas as pl
from jax.experimental.pallas import tpu as pltpu
from jax.experimental.pallas import tpu_sc as plsc

N = 100000
E = 1600000
IN_DIM = 4
HID = 64

NC = 2
NS = 16
NW = NC * NS

N_PAD = 100352
NSL = N_PAD // NS
SAC = N

EPT = 50048
E_PAD = NW * EPT
CHE = 3128
NCHUNK = EPT // CHE

f32 = jnp.float32


def _worker_ids():
    cid = lax.axis_index("c")
    sid = lax.axis_index("s")
    return cid, sid, cid * NS + sid


def _sc_deg_body(srcR, dstR, zeros_n, ones2,
                 degf_out, degu_out,
                 degf_sh, degu_sh,
                 s_v0, s_v1, s_v2, d_v0, d_v1, d_v2, ones_v,
                 sem_i, sem_s0, sem_s1, sem_s2, sem_st):
    s_v = (s_v0, s_v1, s_v2)
    d_v = (d_v0, d_v1, d_v2)
    sem_s = (sem_s0, sem_s1, sem_s2)
    cid, sid, wid = _worker_ids()
    off = sid * NSL
    sl = pl.ds(off, NSL)
    stage = [pltpu.async_copy(zeros_n.at[sl], degf_sh.at[sl], sem_st),
             pltpu.async_copy(zeros_n.at[sl], degu_sh.at[sl], sem_st),
             pltpu.async_copy(ones2, ones_v, sem_st)]
    for d in stage:
        d.wait()
    plsc.subcore_barrier()
    e0 = wid * EPT

    idx_pend = [None] * 3
    sca_pend = [None] * 3

    def start_idx(c):
        r = e0 + c * CHE
        m = c % 3
        idx_pend[m] = [
            pltpu.async_copy(srcR.at[pl.ds(r, CHE)], s_v[m], sem_i),
            pltpu.async_copy(dstR.at[pl.ds(r, CHE)], d_v[m], sem_i)]

    start_idx(0)
    for c in range(NCHUNK):
        m = c % 3
        for d in idx_pend[m]:
            d.wait()
        if c >= 2 and sca_pend[(c - 2) % 3]:
            for d in sca_pend[(c - 2) % 3]:
                d.wait()
            sca_pend[(c - 2) % 3] = None
        if c + 1 < NCHUNK:
            start_idx(c + 1)
        sca_pend[m] = []
        for dstref, idxref in ((degf_sh, d_v[m]), (degu_sh, s_v[m])):
            dd = pltpu.make_async_copy(ones_v, dstref.at[idxref], sem_s[m])
            dd.start(add=True)
            sca_pend[m].append(dd)
    for m in range(3):
        if sca_pend[m]:
            for d in sca_pend[m]:
                d.wait()
    plsc.subcore_barrier()
    out = [pltpu.async_copy(degf_sh.at[sl], degf_out.at[cid, sl], sem_st),
           pltpu.async_copy(degu_sh.at[sl], degu_out.at[cid, sl], sem_st)]
    for d in out:
        d.wait()


_sc_deg = pl.kernel(
    _sc_deg_body,
    out_type=[jax.ShapeDtypeStruct((NC, N_PAD), f32),
              jax.ShapeDtypeStruct((NC, N_PAD), f32)],
    mesh=plsc.VectorSubcoreMesh(core_axis_name="c", subcore_axis_name="s"),
    scratch_types=(
        [pltpu.VMEM_SHARED((N_PAD,), f32)] * 2
        + [pltpu.VMEM((CHE,), jnp.int32)] * 6
        + [pltpu.VMEM((CHE,), f32)]
        + [pltpu.SemaphoreType.DMA] * 5
    ),
)


def _sc_agg_body(aR, bR, tT_hbm, zeros_n,
                 acc_out,
                 t_sh0, t_sh1, t_sh2, t_sh3,
                 acc_sh0, acc_sh1, acc_sh2, acc_sh3,
                 a_v0, a_v1, a_v2, b_v0, b_v1, b_v2,
                 v00, v01, v02, v03, v10, v11, v12, v13,
                 sem_i, sem_g, sem_s0, sem_s1, sem_st):
    t_sh = (t_sh0, t_sh1, t_sh2, t_sh3)
    acc_sh = (acc_sh0, acc_sh1, acc_sh2, acc_sh3)
    a_v = (a_v0, a_v1, a_v2)
    b_v = (b_v0, b_v1, b_v2)
    val = ((v00, v01, v02, v03), (v10, v11, v12, v13))
    sem_s = (sem_s0, sem_s1)
    cid, sid, wid = _worker_ids()
    off = sid * NSL
    sl = pl.ds(off, NSL)
    stage = []
    for k in range(IN_DIM):
        stage.append(pltpu.async_copy(tT_hbm.at[k, sl], t_sh[k].at[sl], sem_st))
        stage.append(pltpu.async_copy(zeros_n.at[sl], acc_sh[k].at[sl], sem_st))
    for d in stage:
        d.wait()
    plsc.subcore_barrier()
    e0 = wid * EPT

    idx_pend = [None] * 3
    sca_pend = [None, None]

    def start_idx(c):
        r = e0 + c * CHE
        m = c % 3
        idx_pend[m] = [
            pltpu.async_copy(aR.at[pl.ds(r, CHE)], a_v[m], sem_i),
            pltpu.async_copy(bR.at[pl.ds(r, CHE)], b_v[m], sem_i)]

    start_idx(0)
    for c in range(NCHUNK):
        m = c % 3
        p = c & 1
        for d in idx_pend[m]:
            d.wait()
        if sca_pend[p]:
            for d in sca_pend[p]:
                d.wait()
            sca_pend[p] = None
        if c + 1 < NCHUNK:
            start_idx(c + 1)
        gat = [pltpu.async_copy(t_sh[k].at[a_v[m]], val[p][k], sem_g)
               for k in range(IN_DIM)]
        for d in gat:
            d.wait()
        sca_pend[p] = []
        for k in range(IN_DIM):
            dd = pltpu.make_async_copy(val[p][k], acc_sh[k].at[b_v[m]], sem_s[p])
            dd.start(add=True)
            sca_pend[p].append(dd)
    for p in (0, 1):
        if sca_pend[p]:
            for d in sca_pend[p]:
                d.wait()
    plsc.subcore_barrier()
    out = [pltpu.async_copy(acc_sh[k].at[sl], acc_out.at[cid, k, sl], sem_st)
           for k in range(IN_DIM)]
    for d in out:
        d.wait()


_sc_agg = pl.kernel(
    _sc_agg_body,
    out_type=jax.ShapeDtypeStruct((NC, IN_DIM, N_PAD), f32),
    mesh=plsc.VectorSubcoreMesh(core_axis_name="c", subcore_axis_name="s"),
    scratch_types=(
        [pltpu.VMEM_SHARED((N_PAD,), f32)] * 8
        + [pltpu.VMEM((CHE,), jnp.int32)] * 6
        + [pltpu.VMEM((CHE,), f32)] * 8
        + [pltpu.SemaphoreType.DMA] * 5
    ),
)


def _sc_y_body(srcR, dstR, ys_hbm, zeros_n,
               ay_out,
               ys_sh, ay_sh,
               s_v0, s_v1, s_v2, d_v0, d_v1, d_v2, r1_v0, r1_v1,
               sem_i, sem_g, sem_s0, sem_s1, sem_st):
    s_v = (s_v0, s_v1, s_v2)
    d_v = (d_v0, d_v1, d_v2)
    r1 = (r1_v0, r1_v1)
    sem_s = (sem_s0, sem_s1)
    cid, sid, wid = _worker_ids()
    off = sid * NSL
    sl = pl.ds(off, NSL)
    stage = [pltpu.async_copy(ys_hbm.at[sl], ys_sh.at[sl], sem_st),
             pltpu.async_copy(zeros_n.at[sl], ay_sh.at[sl], sem_st)]
    for d in stage:
        d.wait()
    plsc.subcore_barrier()
    e0 = wid * EPT

    idx_pend = [None] * 3
    sca_pend = [None, None]

    def start_idx(c):
        r = e0 + c * CHE
        m = c % 3
        idx_pend[m] = [
            pltpu.async_copy(srcR.at[pl.ds(r, CHE)], s_v[m], sem_i),
            pltpu.async_copy(dstR.at[pl.ds(r, CHE)], d_v[m], sem_i)]

    start_idx(0)
    for c in range(NCHUNK):
        m = c % 3
        p = c & 1
        for d in idx_pend[m]:
            d.wait()
        if sca_pend[p]:
            for d in sca_pend[p]:
                d.wait()
            sca_pend[p] = None
        if c + 1 < NCHUNK:
            start_idx(c + 1)
        pltpu.async_copy(ys_sh.at[s_v[m]], r1[p], sem_g).wait()
        dd = pltpu.make_async_copy(r1[p], ay_sh.at[d_v[m]], sem_s[p])
        dd.start(add=True)
        sca_pend[p] = [dd]
    for p in (0, 1):
        if sca_pend[p]:
            for d in sca_pend[p]:
                d.wait()
    plsc.subcore_barrier()
    pltpu.async_copy(ay_sh.at[sl], ay_out.at[cid, sl], sem_st).wait()


_sc_y = pl.kernel(
    _sc_y_body,
    out_type=jax.ShapeDtypeStruct((NC, N_PAD), f32),
    mesh=plsc.VectorSubcoreMesh(core_axis_name="c", subcore_axis_name="s"),
    scratch_types=(
        [pltpu.VMEM_SHARED((N_PAD,), f32)] * 2
        + [pltpu.VMEM((CHE,), jnp.int32)] * 6
        + [pltpu.VMEM((CHE,), f32)] * 2
        + [pltpu.SemaphoreType.DMA] * 5
    ),
)


def _tc_norm_body(degf_ref, degu_ref, xT_ref,
                  df_ref, du_ref, pT_ref, qT_ref):
    degf = degf_ref[0:1, :] + degf_ref[1:2, :] + 1.0
    degu = degu_ref[0:1, :] + degu_ref[1:2, :] + 1.0
    df = 1.0 / jnp.sqrt(degf)
    du = 1.0 / jnp.sqrt(degu)
    df_ref[...] = df
    du_ref[...] = du
    pT_ref[...] = xT_ref[...] * df
    qT_ref[...] = xT_ref[...] * du


_tc_norm = pl.pallas_call(
    _tc_norm_body,
    out_shape=[jax.ShapeDtypeStruct((1, N_PAD), f32),
               jax.ShapeDtypeStruct((1, N_PAD), f32),
               jax.ShapeDtypeStruct((IN_DIM, N_PAD), f32),
               jax.ShapeDtypeStruct((IN_DIM, N_PAD), f32)],
)

BN = 2048
GRID_B = N_PAD // BN


def _mm4(wT, a):
    acc = wT[:, 0:1] * a[0:1, :]
    for k in range(1, IN_DIM):
        acc += wT[:, k:k + 1] * a[k:k + 1, :]
    return acc


def _tc_dense_body(afT_ref, auT_ref, pT_ref, qT_ref, df_ref, du_ref,
                   wfT_ref, wuT_ref, bf_ref, bu_ref,
                   wfc1T_ref, wfc2T_ref, bfc_ref, wo_ref,
                   ys_ref):
    af = (afT_ref[0] + afT_ref[1] + pT_ref[...]) * df_ref[...]
    au = (auT_ref[0] + auT_ref[1] + qT_ref[...]) * du_ref[...]
    hf = jnp.maximum(_mm4(wfT_ref[...], af) + bf_ref[...], 0.0)
    hu = jnp.maximum(_mm4(wuT_ref[...], au) + bu_ref[...], 0.0)
    h2 = jnp.dot(wfc1T_ref[...], hf, preferred_element_type=f32)
    h2 += jnp.dot(wfc2T_ref[...], hu, preferred_element_type=f32)
    h2 = jnp.maximum(h2 + bfc_ref[...], 0.0)
    y = jnp.sum(h2 * wo_ref[...], axis=0, keepdims=True)
    ys_ref[...] = y * df_ref[...]


_tc_dense = pl.pallas_call(
    _tc_dense_body,
    grid=(N_PAD // BN,),
    in_specs=[
        pl.BlockSpec((NC, IN_DIM, BN), lambda i: (0, 0, i)),
        pl.BlockSpec((NC, IN_DIM, BN), lambda i: (0, 0, i)),
        pl.BlockSpec((IN_DIM, BN), lambda i: (0, i)),
        pl.BlockSpec((IN_DIM, BN), lambda i: (0, i)),
        pl.BlockSpec((1, BN), lambda i: (0, i)),
        pl.BlockSpec((1, BN), lambda i: (0, i)),
        pl.BlockSpec((HID, IN_DIM), lambda i: (0, 0)),
        pl.BlockSpec((HID, IN_DIM), lambda i: (0, 0)),
        pl.BlockSpec((HID, 1), lambda i: (0, 0)),
        pl.BlockSpec((HID, 1), lambda i: (0, 0)),
        pl.BlockSpec((HID, HID), lambda i: (0, 0)),
        pl.BlockSpec((HID, HID), lambda i: (0, 0)),
        pl.BlockSpec((HID, 1), lambda i: (0, 0)),
        pl.BlockSpec((HID, 1), lambda i: (0, 0)),
    ],
    out_specs=pl.BlockSpec((1, BN), lambda i: (0, i)),
    out_shape=jax.ShapeDtypeStruct((1, N_PAD), f32),
)


def _tc_out_body(ay_ref, ys_ref, df_ref, bo_ref, out_ref):
    s = df_ref[...] * (ay_ref[0:1, :] + ay_ref[1:2, :] + ys_ref[...]) + bo_ref[0, 0]
    out_ref[...] = jax.nn.sigmoid(s)


_tc_out = pl.pallas_call(
    _tc_out_body,
    out_shape=jax.ShapeDtypeStruct((1, N_PAD), f32),
)


@jax.jit
def kernel(x, edge_index, W_f, b_f, W_u, b_u, W_fc, b_fc, W_o, b_o):
    src = edge_index[0].astype(jnp.int32)
    dst = edge_index[1].astype(jnp.int32)
    padi = jnp.full((E_PAD - E,), SAC, dtype=jnp.int32)
    srcR = jnp.concatenate([src, padi])
    dstR = jnp.concatenate([dst, padi])

    zeros_n = jnp.zeros((N_PAD,), f32)
    ones2 = jnp.ones((CHE,), f32)

    degf_p, degu_p = _sc_deg(srcR, dstR, zeros_n, ones2)

    xT = jnp.zeros((IN_DIM, N_PAD), f32).at[:, :N].set(x.T)
    df, du, pT, qT = _tc_norm(degf_p, degu_p, xT)

    af_p = _sc_agg(srcR, dstR, pT, zeros_n)
    au_p = _sc_agg(dstR, srcR, qT, zeros_n)

    ysT = _tc_dense(af_p, au_p, pT, qT, df, du,
                    W_f.T, W_u.T, b_f[:, None], b_u[:, None],
                    W_fc[:HID].T, W_fc[HID:].T, b_fc[:, None], W_o)

    ay_p = _sc_y(srcR, dstR, ysT[0], zeros_n)

    outT = _tc_out(ay_p, ysT, df, b_o.reshape(1, 1))
    return outT[0, :N][:, None]
